# baseline (speedup 1.0000x reference)
import functools

import jax
import jax.numpy as jnp
from jax import lax
from jax.experimental import pallas as pl
from jax.experimental.pallas import tpu as pltpu
from jax.experimental.pallas import tpu_sc as plsc

N = 100000
D = 128
G = 512

NC = 2
NS = 16
NW = NC * NS

ROWS_PER_STEP = 128
STEPS1 = 14
STEPS2 = 11
CH1 = NW * ROWS_PER_STEP * STEPS1
CH2 = NW * ROWS_PER_STEP * STEPS2
N_PAD = CH1 + CH2

ACC_ROWS = G + 2 * NS
ZROWS = ACC_ROWS // NS

MM_BN = 4096
MM_XBLOCKS = -(-N // MM_BN) - 1


def _mm_body(x_ref, we_ref, be_ref, wb_ref, bb_ref, out_ref):
    h = jnp.dot(x_ref[...], we_ref[...], preferred_element_type=jnp.float32)
    h = jnp.maximum(h + be_ref[...], 0.0)
    out_ref[...] = jnp.dot(h, wb_ref[...],
                           preferred_element_type=jnp.float32) + bb_ref[...]


def _node_features(x, W_enc, b_enc, W_bb, b_bb, rows, block0):
    return pl.pallas_call(
        _mm_body,
        grid=(rows // MM_BN,),
        in_specs=[
            pl.BlockSpec((MM_BN, D),
                         lambda i: (jnp.minimum(block0 + i, MM_XBLOCKS), 0)),
            pl.BlockSpec((D, D), lambda i: (0, 0)),
            pl.BlockSpec((1, D), lambda i: (0, 0)),
            pl.BlockSpec((D, D), lambda i: (0, 0)),
            pl.BlockSpec((1, D), lambda i: (0, 0)),
        ],
        out_specs=pl.BlockSpec((MM_BN, D), lambda i: (i, 0)),
        out_shape=jax.ShapeDtypeStruct((rows, D), jnp.float32),
    )(x, W_enc, b_enc.reshape(1, D), W_bb, b_bb.reshape(1, D))


def _zero_acc(zero_v, acc_sh, sid):
    zeros16 = jnp.zeros((16,), jnp.float32)
    for r in range(ZROWS):
        for c in range(D // 16):
            zero_v[r, pl.ds(c * 16, 16)] = zeros16
    pltpu.sync_copy(zero_v, acc_sh.at[pl.ds(sid * ZROWS, ZROWS)])


def _write_partial(acc_sh, part_hbm, cid, sid):
    rows_out = G // NS
    pltpu.sync_copy(acc_sh.at[pl.ds(sid * rows_out, rows_out)],
                    part_hbm.at[cid, pl.ds(sid * rows_out, rows_out)])


def _make_sc_body(steps):
    def _sc_body(nf_hbm, idx_hbm, part_hbm, cnt_hbm,
                 idx_v, row_v0, row_v1, zero_v, cnt_v,
                 sem0, sem1, sem_s, acc_sh):
        cid = lax.axis_index("c")
        sid = lax.axis_index("s")
        wid = sid * NC + cid

        zeros16 = jnp.zeros((16,), jnp.float32)
        for r in range(ACC_ROWS // 16):
            cnt_v[pl.ds(r * 16, 16)] = zeros16
        _zero_acc(zero_v, acc_sh, sid)
        plsc.subcore_barrier()

        pltpu.sync_copy(idx_hbm.at[wid], idx_v)

        e0 = jnp.where(lax.iota(jnp.int32, 16) == 0, 1.0, 0.0)

        def _hist_step(j):
            def _hist16(c, carry):
                ids16 = idx_v[j, pl.ds(c * 16, 16)]
                for t in range(16):
                    plsc.addupdate(cnt_v.at[pl.ds(ids16[t], 16)], e0)
                return carry
            lax.fori_loop(0, ROWS_PER_STEP // 16, _hist16, 0)

        base = wid * steps * ROWS_PER_STEP
        bufs = (row_v0, row_v1)
        sems = (sem0, sem1)
        handles = [None, None]
        handles[0] = pltpu.async_copy(
            nf_hbm.at[pl.ds(base, ROWS_PER_STEP)], row_v0, sem0)
        for j in range(steps):
            b = bufs[j % 2]
            handles[j % 2].wait()
            sc_h = pltpu.async_copy(b, acc_sh.at[idx_v.at[j]], sem_s,
                                    add=True)
            if j + 1 < steps:
                handles[(j + 1) % 2] = pltpu.async_copy(
                    nf_hbm.at[pl.ds(base + (j + 1) * ROWS_PER_STEP,
                                    ROWS_PER_STEP)],
                    bufs[(j + 1) % 2], sems[(j + 1) % 2])
            _hist_step(j)
            sc_h.wait()

        plsc.subcore_barrier()
        _write_partial(acc_sh, part_hbm, cid, sid)
        pltpu.sync_copy(cnt_v, cnt_hbm.at[wid])
    return _sc_body


def _segment_partials(nf, idx, steps):
    mesh = plsc.VectorSubcoreMesh(core_axis_name="c", subcore_axis_name="s",
                                  num_cores=NC, num_subcores=NS)
    k = functools.partial(
        pl.kernel,
        out_type=[jax.ShapeDtypeStruct((NC, G, D), jnp.float32),
                  jax.ShapeDtypeStruct((NW, ACC_ROWS), jnp.float32)],
        mesh=mesh,
        scratch_types=[
            pltpu.VMEM((steps, ROWS_PER_STEP), jnp.int32),
            pltpu.VMEM((ROWS_PER_STEP, D), jnp.float32),
            pltpu.VMEM((ROWS_PER_STEP, D), jnp.float32),
            pltpu.VMEM((ZROWS, D), jnp.float32),
            pltpu.VMEM((ACC_ROWS,), jnp.float32),
            pltpu.SemaphoreType.DMA,
            pltpu.SemaphoreType.DMA,
            pltpu.SemaphoreType.DMA,
            pltpu.VMEM_SHARED((ACC_ROWS, D), jnp.float32),
        ],
    )(_make_sc_body(steps))
    return k(nf, idx)


def _comb_body(p1_ref, p2_ref, c1_ref, c2_ref, out_ref):
    s = p1_ref[0] + p1_ref[1] + p2_ref[0] + p2_ref[1]
    crow = (jnp.sum(c1_ref[...], axis=0, keepdims=True)
            + jnp.sum(c2_ref[...], axis=0, keepdims=True))
    cnt = jnp.transpose(crow)[0:G]
    out_ref[...] = s / jnp.maximum(cnt, 1.0)


def _combine(p1, p2, c1, c2):
    return pl.pallas_call(
        _comb_body,
        out_shape=jax.ShapeDtypeStruct((G, D), jnp.float32),
    )(p1, p2, c1, c2)


def kernel(x, batch_0, W_enc, b_enc, W_bb, b_bb):
    idx_pad = jnp.concatenate(
        [batch_0, jnp.full((N_PAD - N,), G, jnp.int32)])
    idx1 = idx_pad[:CH1].reshape(NW, STEPS1, ROWS_PER_STEP)
    idx2 = idx_pad[CH1:].reshape(NW, STEPS2, ROWS_PER_STEP)
    nf1 = _node_features(x, W_enc, b_enc, W_bb, b_bb, CH1, 0)
    p1, c1 = _segment_partials(nf1, idx1, STEPS1)
    nf2 = _node_features(x, W_enc, b_enc, W_bb, b_bb, CH2, CH1 // MM_BN)
    p2, c2 = _segment_partials(nf2, idx2, STEPS2)
    return _combine(p1, p2, c1, c2)


# speedup vs baseline: 1.6570x; 1.6570x over previous
import functools

import jax
import jax.numpy as jnp
from jax import lax
from jax.experimental import pallas as pl
from jax.experimental.pallas import tpu as pltpu
from jax.experimental.pallas import tpu_sc as plsc

N = 100000
D = 128
G = 512

NC = 2
NS = 16
NW = NC * NS

ROWS_PER_STEP = 128
STEPS1 = 13
STEPS2 = 12
CH1 = NW * ROWS_PER_STEP * STEPS1
CH2 = NW * ROWS_PER_STEP * STEPS2
N_PAD = CH1 + CH2

ACC_ROWS = G + 2 * NS
ZROWS = ACC_ROWS // NS

MM_BN = 4096
MM_XBLOCKS = -(-N // MM_BN) - 1


def _mm_body(x_ref, we_ref, be_ref, wb_ref, bb_ref, out_ref):
    h = jnp.dot(x_ref[...], we_ref[...], preferred_element_type=jnp.float32)
    h = jnp.maximum(h + be_ref[...], 0.0)
    out_ref[...] = jnp.dot(h, wb_ref[...],
                           preferred_element_type=jnp.float32) + bb_ref[...]


def _node_features(x, W_enc, b_enc, W_bb, b_bb, rows, block0):
    return pl.pallas_call(
        _mm_body,
        grid=(rows // MM_BN,),
        in_specs=[
            pl.BlockSpec((MM_BN, D),
                         lambda i: (jnp.minimum(block0 + i, MM_XBLOCKS), 0)),
            pl.BlockSpec((D, D), lambda i: (0, 0)),
            pl.BlockSpec((1, D), lambda i: (0, 0)),
            pl.BlockSpec((D, D), lambda i: (0, 0)),
            pl.BlockSpec((1, D), lambda i: (0, 0)),
        ],
        out_specs=pl.BlockSpec((MM_BN, D), lambda i: (i, 0)),
        out_shape=jax.ShapeDtypeStruct((rows, D), jnp.float32),
    )(x, W_enc, b_enc.reshape(1, D), W_bb, b_bb.reshape(1, D))


def _zero_acc(zero_v, acc_sh, sid):
    zeros16 = jnp.zeros((16,), jnp.float32)
    for r in range(ZROWS):
        for c in range(D // 16):
            zero_v[r, pl.ds(c * 16, 16)] = zeros16
    pltpu.sync_copy(zero_v, acc_sh.at[pl.ds(sid * ZROWS, ZROWS)])


def _write_partial(acc_sh, part_hbm, cid, sid):
    rows_out = G // NS
    pltpu.sync_copy(acc_sh.at[pl.ds(sid * rows_out, rows_out)],
                    part_hbm.at[cid, pl.ds(sid * rows_out, rows_out)])


def _make_sc_body(steps):
    def _sc_body(nf_hbm, idx_hbm, part_hbm, cnt_hbm,
                 idx_v, row_v0, row_v1, zero_v, cnt_v,
                 sem0, sem1, sem_s, acc_sh):
        cid = lax.axis_index("c")
        sid = lax.axis_index("s")
        wid = sid * NC + cid

        zeros16 = jnp.zeros((16,), jnp.float32)
        for r in range(ACC_ROWS // 16):
            cnt_v[pl.ds(r * 16, 16)] = zeros16
        _zero_acc(zero_v, acc_sh, sid)
        plsc.subcore_barrier()

        pltpu.sync_copy(idx_hbm.at[wid], idx_v)

        e0 = jnp.where(lax.iota(jnp.int32, 16) == 0, 1.0, 0.0)

        def _hist_step(j):
            def _hist16(c, carry):
                ids16 = idx_v[j, pl.ds(c * 16, 16)]
                for t in range(16):
                    plsc.addupdate(cnt_v.at[pl.ds(ids16[t], 16)], e0)
                return carry
            lax.fori_loop(0, ROWS_PER_STEP // 16, _hist16, 0)

        base = wid * steps * ROWS_PER_STEP
        bufs = (row_v0, row_v1)
        sems = (sem0, sem1)
        handles = [None, None]
        handles[0] = pltpu.async_copy(
            nf_hbm.at[pl.ds(base, ROWS_PER_STEP)], row_v0, sem0)
        for j in range(steps):
            b = bufs[j % 2]
            handles[j % 2].wait()
            sc_h = pltpu.async_copy(b, acc_sh.at[idx_v.at[j]], sem_s,
                                    add=True)
            if j + 1 < steps:
                handles[(j + 1) % 2] = pltpu.async_copy(
                    nf_hbm.at[pl.ds(base + (j + 1) * ROWS_PER_STEP,
                                    ROWS_PER_STEP)],
                    bufs[(j + 1) % 2], sems[(j + 1) % 2])
            _hist_step(j)
            sc_h.wait()

        plsc.subcore_barrier()
        _write_partial(acc_sh, part_hbm, cid, sid)
        pltpu.sync_copy(cnt_v, cnt_hbm.at[wid])
    return _sc_body


def _segment_partials(nf, idx, steps):
    mesh = plsc.VectorSubcoreMesh(core_axis_name="c", subcore_axis_name="s",
                                  num_cores=NC, num_subcores=NS)
    k = functools.partial(
        pl.kernel,
        out_type=[jax.ShapeDtypeStruct((NC, G, D), jnp.float32),
                  jax.ShapeDtypeStruct((NW, ACC_ROWS), jnp.float32)],
        mesh=mesh,
        scratch_types=[
            pltpu.VMEM((steps, ROWS_PER_STEP), jnp.int32),
            pltpu.VMEM((ROWS_PER_STEP, D), jnp.float32),
            pltpu.VMEM((ROWS_PER_STEP, D), jnp.float32),
            pltpu.VMEM((ZROWS, D), jnp.float32),
            pltpu.VMEM((ACC_ROWS,), jnp.float32),
            pltpu.SemaphoreType.DMA,
            pltpu.SemaphoreType.DMA,
            pltpu.SemaphoreType.DMA,
            pltpu.VMEM_SHARED((ACC_ROWS, D), jnp.float32),
        ],
    )(_make_sc_body(steps))
    return k(nf, idx)


def _comb_body(p1_ref, p2_ref, c1_ref, c2_ref, out_ref):
    s = p1_ref[0] + p1_ref[1] + p2_ref[0] + p2_ref[1]
    crow = (jnp.sum(c1_ref[...], axis=0, keepdims=True)
            + jnp.sum(c2_ref[...], axis=0, keepdims=True))
    cnt = jnp.transpose(crow)[0:G]
    out_ref[...] = s / jnp.maximum(cnt, 1.0)


def _combine(p1, p2, c1, c2):
    return pl.pallas_call(
        _comb_body,
        out_shape=jax.ShapeDtypeStruct((G, D), jnp.float32),
    )(p1, p2, c1, c2)


def kernel(x, batch_0, W_enc, b_enc, W_bb, b_bb):
    idx_pad = jnp.concatenate(
        [batch_0, jnp.full((N_PAD - N,), G, jnp.int32)])
    idx1 = idx_pad[:CH1].reshape(NW, STEPS1, ROWS_PER_STEP)
    idx2 = idx_pad[CH1:].reshape(NW, STEPS2, ROWS_PER_STEP)
    nf1 = _node_features(x, W_enc, b_enc, W_bb, b_bb, CH1, 0)
    p1, c1 = _segment_partials(nf1, idx1, STEPS1)
    nf2 = _node_features(x, W_enc, b_enc, W_bb, b_bb, CH2, CH1 // MM_BN)
    p2, c2 = _segment_partials(nf2, idx2, STEPS2)
    return _combine(p1, p2, c1, c2)
